# baseline (speedup 1.0000x reference)
import jax
import jax.numpy as jnp
from jax import lax
from jax.experimental import pallas as pl
from jax.experimental.pallas import tpu as pltpu
from jax.experimental.pallas import tpu_sc as plsc

N = 10000
E = 320000
D = 128

NC = 2
NS = 16
NW = NC * NS
EPW = E // NW
CHUNK = 128
NCHUNK = 80
PAD = NCHUNK * CHUNK - EPW
ACC_N = 10112
RPT = ACC_N // NS


def _sc_agg_body(src_hbm, dst_hbm, x_hbm, zeros_hbm, out_hbm,
                 sidx, dst_v, rows_a, rows_b, acc_sh, sem_i, sem_a, sem_b):
    c = lax.axis_index("c")
    s = lax.axis_index("s")
    wid = c * NS + s

    pltpu.sync_copy(zeros_hbm.at[s], acc_sh.at[pl.ds(s * RPT, RPT)])
    pltpu.sync_copy(dst_hbm.at[wid], dst_v)
    plsc.subcore_barrier()

    pltpu.async_copy(src_hbm.at[wid, 0], sidx.at[0], sem_i).wait()
    pltpu.async_copy(x_hbm.at[sidx.at[0]], rows_a, sem_a)
    pltpu.async_copy(src_hbm.at[wid, 1], sidx.at[1], sem_i)

    def body(jj, carry):
        j = 2 * jj
        pltpu.make_async_copy(src_hbm.at[wid, j + 1], sidx.at[1], sem_i).wait()
        pltpu.async_copy(x_hbm.at[sidx.at[1]], rows_b, sem_b)
        pltpu.make_async_copy(x_hbm.at[sidx.at[0]], rows_a, sem_a).wait()
        pltpu.async_copy(src_hbm.at[wid, j + 2], sidx.at[0], sem_i)
        pltpu.sync_copy(rows_a, acc_sh.at[dst_v.at[j]], add=True)
        pltpu.make_async_copy(src_hbm.at[wid, j + 2], sidx.at[0], sem_i).wait()
        pltpu.async_copy(x_hbm.at[sidx.at[0]], rows_a, sem_a)
        pltpu.make_async_copy(x_hbm.at[sidx.at[1]], rows_b, sem_b).wait()
        pltpu.async_copy(src_hbm.at[wid, j + 3], sidx.at[1], sem_i)
        pltpu.sync_copy(rows_b, acc_sh.at[dst_v.at[j + 1]], add=True)
        return carry

    lax.fori_loop(0, NCHUNK // 2 - 1, body, 0)
    j = NCHUNK - 2
    pltpu.make_async_copy(src_hbm.at[wid, j + 1], sidx.at[1], sem_i).wait()
    pltpu.async_copy(x_hbm.at[sidx.at[1]], rows_b, sem_b)
    pltpu.make_async_copy(x_hbm.at[sidx.at[0]], rows_a, sem_a).wait()
    pltpu.sync_copy(rows_a, acc_sh.at[dst_v.at[j]], add=True)
    pltpu.make_async_copy(x_hbm.at[sidx.at[1]], rows_b, sem_b).wait()
    pltpu.sync_copy(rows_b, acc_sh.at[dst_v.at[j + 1]], add=True)
    plsc.subcore_barrier()

    pltpu.sync_copy(acc_sh.at[pl.ds(s * RPT, RPT)], out_hbm.at[c, s])


@jax.jit
def _sc_agg(src3d, dst3d, x, zeros):
    mesh = plsc.VectorSubcoreMesh(core_axis_name="c", subcore_axis_name="s",
                                  num_cores=NC, num_subcores=NS)
    f = pl.kernel(
        _sc_agg_body,
        out_type=jax.ShapeDtypeStruct((NC, NS, RPT, D), jnp.float32),
        mesh=mesh,
        scratch_types=[
            pltpu.VMEM((2, CHUNK), jnp.int32),
            pltpu.VMEM((NCHUNK, CHUNK), jnp.int32),
            pltpu.VMEM((CHUNK, D), jnp.float32),
            pltpu.VMEM((CHUNK, D), jnp.float32),
            pltpu.VMEM_SHARED((ACC_N, D), jnp.float32),
            pltpu.SemaphoreType.DMA,
            pltpu.SemaphoreType.DMA,
            pltpu.SemaphoreType.DMA,
        ],
    )
    return f(src3d, dst3d, x, zeros)


def _tc_mlp_body(x_ref, p_ref, W1_ref, b1_ref, W2_ref, b2_ref, W3_ref, b3_ref,
                 gamma_ref, beta_ref, o_ref):
    h = x_ref[...] + p_ref[0, :N] + p_ref[1, :N]
    h = jax.nn.sigmoid(
        jnp.dot(h, W1_ref[...], preferred_element_type=jnp.float32)
        + b1_ref[...])
    h = jax.nn.sigmoid(
        jnp.dot(h, W2_ref[...], preferred_element_type=jnp.float32)
        + b2_ref[...])
    h = (jnp.dot(h, W3_ref[...], preferred_element_type=jnp.float32)
         + b3_ref[...])
    h = jnp.where(h >= 0, h, 0.01 * h)
    mean = jnp.mean(h, axis=0, keepdims=True)
    var = jnp.mean(h * h, axis=0, keepdims=True) - mean * mean
    o_ref[...] = ((h - mean) * jax.lax.rsqrt(var + 1e-5) * gamma_ref[...]
                  + beta_ref[...])


@jax.jit
def _tc_mlp(x, partials, W1, b1, W2, b2, W3, b3, gamma, beta):
    return pl.pallas_call(
        _tc_mlp_body,
        out_shape=jax.ShapeDtypeStruct((N, D), jnp.float32),
    )(x, partials, W1, b1.reshape(1, -1), W2, b2.reshape(1, -1),
      W3, b3.reshape(1, -1), gamma.reshape(1, -1), beta.reshape(1, -1))


@jax.jit
def kernel(x, edge_index, W1, b1, W2, b2, W3, b3, gamma, beta):
    src = edge_index[0].reshape(NW, EPW)
    dst = edge_index[1].reshape(NW, EPW)
    src3d = jnp.pad(src, ((0, 0), (0, PAD))).reshape(NW, NCHUNK, CHUNK)
    dst3d = jnp.pad(dst, ((0, 0), (0, PAD)),
                    constant_values=N).reshape(NW, NCHUNK, CHUNK)
    zeros = jnp.zeros((NS, RPT, D), jnp.float32)
    out4d = _sc_agg(src3d, dst3d, x, zeros)
    partials = out4d.reshape(NC, ACC_N, D)
    h = _tc_mlp(x, partials, W1, b1, W2, b2, W3, b3, gamma, beta)
    return (h, edge_index)


# speedup vs baseline: 1.0001x; 1.0001x over previous
import jax
import jax.numpy as jnp
from jax import lax
from jax.experimental import pallas as pl
from jax.experimental.pallas import tpu as pltpu
from jax.experimental.pallas import tpu_sc as plsc

N = 10000
E = 320000
D = 128

NC = 2
NS = 16
NW = NC * NS
EPW = E // NW
CHUNK = 128
NCHUNK = 80
PAD = NCHUNK * CHUNK - EPW
ACC_N = 10112
RPT = ACC_N // NS
L = 16
SHIFT = 14
MASK = (1 << SHIFT) - 1


def _sc_agg_body(pk_hbm, x_hbm, zeros_hbm, out_hbm,
                 pk_v, sidx, didx, rows_a, rows_b, acc_sh, sem_a, sem_b):
    c = lax.axis_index("c")
    s = lax.axis_index("s")
    wid = c * NS + s

    pltpu.sync_copy(zeros_hbm.at[s], acc_sh.at[pl.ds(s * RPT, RPT)])
    pltpu.sync_copy(pk_hbm.at[wid], pk_v)
    plsc.subcore_barrier()

    def unpack(j, slot):
        for k in range(CHUNK // L):
            pk = pk_v[j, pl.ds(L * k, L)]
            sidx[slot, pl.ds(L * k, L)] = pk & MASK
            didx[slot, pl.ds(L * k, L)] = lax.shift_right_logical(pk, SHIFT)

    unpack(0, 0)
    pltpu.async_copy(x_hbm.at[sidx.at[0]], rows_a, sem_a)
    unpack(1, 1)

    def body(jj, carry):
        j = 2 * jj
        pltpu.async_copy(x_hbm.at[sidx.at[1]], rows_b, sem_b)
        pltpu.make_async_copy(x_hbm.at[sidx.at[0]], rows_a, sem_a).wait()
        pltpu.sync_copy(rows_a, acc_sh.at[didx.at[0]], add=True)
        unpack(j + 2, 0)
        pltpu.async_copy(x_hbm.at[sidx.at[0]], rows_a, sem_a)
        pltpu.make_async_copy(x_hbm.at[sidx.at[1]], rows_b, sem_b).wait()
        pltpu.sync_copy(rows_b, acc_sh.at[didx.at[1]], add=True)
        unpack(j + 3, 1)
        return carry

    lax.fori_loop(0, NCHUNK // 2 - 1, body, 0)
    pltpu.async_copy(x_hbm.at[sidx.at[1]], rows_b, sem_b)
    pltpu.make_async_copy(x_hbm.at[sidx.at[0]], rows_a, sem_a).wait()
    pltpu.sync_copy(rows_a, acc_sh.at[didx.at[0]], add=True)
    pltpu.make_async_copy(x_hbm.at[sidx.at[1]], rows_b, sem_b).wait()
    pltpu.sync_copy(rows_b, acc_sh.at[didx.at[1]], add=True)
    plsc.subcore_barrier()

    pltpu.sync_copy(acc_sh.at[pl.ds(s * RPT, RPT)], out_hbm.at[c, s])


@jax.jit
def _sc_agg(packed3d, x, zeros):
    mesh = plsc.VectorSubcoreMesh(core_axis_name="c", subcore_axis_name="s",
                                  num_cores=NC, num_subcores=NS)
    f = pl.kernel(
        _sc_agg_body,
        out_type=jax.ShapeDtypeStruct((NC, NS, RPT, D), jnp.float32),
        mesh=mesh,
        scratch_types=[
            pltpu.VMEM((NCHUNK, CHUNK), jnp.int32),
            pltpu.VMEM((2, CHUNK), jnp.int32),
            pltpu.VMEM((2, CHUNK), jnp.int32),
            pltpu.VMEM((CHUNK, D), jnp.float32),
            pltpu.VMEM((CHUNK, D), jnp.float32),
            pltpu.VMEM_SHARED((ACC_N, D), jnp.float32),
            pltpu.SemaphoreType.DMA,
            pltpu.SemaphoreType.DMA,
        ],
    )
    return f(packed3d, x, zeros)


def _tc_mlp_body(x_ref, p_ref, W1_ref, b1_ref, W2_ref, b2_ref, W3_ref, b3_ref,
                 gamma_ref, beta_ref, o_ref):
    h = x_ref[...] + p_ref[0, :N] + p_ref[1, :N]
    h = jax.nn.sigmoid(
        jnp.dot(h, W1_ref[...], preferred_element_type=jnp.float32)
        + b1_ref[...])
    h = jax.nn.sigmoid(
        jnp.dot(h, W2_ref[...], preferred_element_type=jnp.float32)
        + b2_ref[...])
    h = (jnp.dot(h, W3_ref[...], preferred_element_type=jnp.float32)
         + b3_ref[...])
    h = jnp.where(h >= 0, h, 0.01 * h)
    mean = jnp.mean(h, axis=0, keepdims=True)
    var = jnp.mean(h * h, axis=0, keepdims=True) - mean * mean
    o_ref[...] = ((h - mean) * jax.lax.rsqrt(var + 1e-5) * gamma_ref[...]
                  + beta_ref[...])


@jax.jit
def _tc_mlp(x, partials, W1, b1, W2, b2, W3, b3, gamma, beta):
    return pl.pallas_call(
        _tc_mlp_body,
        out_shape=jax.ShapeDtypeStruct((N, D), jnp.float32),
    )(x, partials, W1, b1.reshape(1, -1), W2, b2.reshape(1, -1),
      W3, b3.reshape(1, -1), gamma.reshape(1, -1), beta.reshape(1, -1))


@jax.jit
def kernel(x, edge_index, W1, b1, W2, b2, W3, b3, gamma, beta):
    src = edge_index[0].reshape(NW, EPW)
    dst = edge_index[1].reshape(NW, EPW)
    srcp = jnp.pad(src, ((0, 0), (0, PAD)))
    dstp = jnp.pad(dst, ((0, 0), (0, PAD)), constant_values=N)
    packed3d = (srcp + (dstp << SHIFT)).reshape(NW, NCHUNK, CHUNK)
    zeros = jnp.zeros((NS, RPT, D), jnp.float32)
    out4d = _sc_agg(packed3d, x, zeros)
    partials = out4d.reshape(NC, ACC_N, D)
    h = _tc_mlp(x, partials, W1, b1, W2, b2, W3, b3, gamma, beta)
    return (h, edge_index)
